# Initial kernel scaffold; baseline (speedup 1.0000x reference)
import jax
import jax.numpy as jnp
from jax.experimental import pallas as pl


def kernel(x, emb):
    raise NotImplementedError("write your pallas kernel here")


# speedup vs baseline: 2.7413x; 2.7413x over previous
import jax
import jax.numpy as jnp
from jax.experimental import pallas as pl


def _copy_block(emb_ref, out_ref):
    out_ref[...] = emb_ref[...][None]


def kernel(x, emb):
    seq = x.shape[1]
    dim = emb.shape[1]
    block = 512
    grid = (seq // block,)
    return pl.pallas_call(
        _copy_block,
        grid=grid,
        in_specs=[pl.BlockSpec((block, dim), lambda i: (i, 0))],
        out_specs=pl.BlockSpec((1, block, dim), lambda i: (0, i, 0)),
        out_shape=jax.ShapeDtypeStruct((1, seq, dim), emb.dtype),
    )(emb)
